# baseline (speedup 1.0000x reference)
import functools

import jax
import jax.numpy as jnp
from jax import lax
from jax.experimental import pallas as pl
from jax.experimental.pallas import tpu as pltpu
from jax.experimental.pallas import tpu_sc as plsc

VOCAB = 1000000
DIM = 64
B = 4096
N = 200
TOT = B * N
NC, NS = 2, 16
NW = NC * NS
PER_W = TOT // NW
CHUNK = 800
NCHUNK = PER_W // CHUNK

_mesh = plsc.VectorSubcoreMesh(core_axis_name="c", subcore_axis_name="s")


@functools.partial(
    pl.kernel,
    out_type=jax.ShapeDtypeStruct((TOT, DIM), jnp.float32),
    mesh=_mesh,
    compiler_params=pltpu.CompilerParams(use_tc_tiling_on_sc=False),
    scratch_types=[
        pltpu.VMEM((CHUNK,), jnp.int32),
        pltpu.VMEM((CHUNK, DIM), jnp.float32),
        pltpu.SemaphoreType.DMA,
    ],
)
def _emb_lookup(idx_hbm, table_hbm, out_hbm, idx_v, rows_v, sem):
    wid = lax.axis_index("s") * NC + lax.axis_index("c")
    base = wid * PER_W

    def body(k, carry):
        off = base + k * CHUNK
        pltpu.sync_copy(idx_hbm.at[pl.ds(off, CHUNK)], idx_v)
        pltpu.async_copy(table_hbm.at[idx_v], rows_v, sem).wait()
        pltpu.sync_copy(rows_v, out_hbm.at[pl.ds(off, CHUNK)])
        return carry

    lax.fori_loop(0, NCHUNK, body, 0)


def kernel(input_ids, table):
    flat = input_ids.reshape(TOT).astype(jnp.int32)
    out = _emb_lookup(flat, table)
    return out.reshape(B, N, DIM)


# speedup vs baseline: 1.0247x; 1.0247x over previous
import functools

import jax
import jax.numpy as jnp
from jax import lax
from jax.experimental import pallas as pl
from jax.experimental.pallas import tpu as pltpu
from jax.experimental.pallas import tpu_sc as plsc

VOCAB = 1000000
DIM = 64
B = 4096
N = 200
TOT = B * N
NC, NS = 2, 16
NW = NC * NS
PER_W = TOT // NW
CHUNK = 400
NCHUNK = PER_W // CHUNK
NBUF = 4

_mesh = plsc.VectorSubcoreMesh(core_axis_name="c", subcore_axis_name="s")


@functools.partial(
    pl.kernel,
    out_type=jax.ShapeDtypeStruct((TOT, DIM), jnp.float32),
    mesh=_mesh,
    compiler_params=pltpu.CompilerParams(use_tc_tiling_on_sc=False),
    scratch_types=[
        pltpu.VMEM((PER_W,), jnp.int32),
        [pltpu.VMEM((CHUNK, DIM), jnp.float32) for _ in range(NBUF)],
        [pltpu.SemaphoreType.DMA for _ in range(NBUF)],
        [pltpu.SemaphoreType.DMA for _ in range(NBUF)],
        pltpu.SemaphoreType.DMA,
    ],
)
def _emb_lookup(idx_hbm, table_hbm, out_hbm, idx_v, rows, gsem, wsem, isem):
    wid = lax.axis_index("s") * NC + lax.axis_index("c")
    base = wid * PER_W

    pltpu.async_copy(idx_hbm.at[pl.ds(base, PER_W)], idx_v, isem).wait()

    def start_gather(k, b):
        pltpu.async_copy(
            table_hbm.at[idx_v.at[pl.ds(k * CHUNK, CHUNK)]], rows[b], gsem[b])

    def wait_gather(k, b):
        pltpu.make_async_copy(
            table_hbm.at[idx_v.at[pl.ds(k * CHUNK, CHUNK)]], rows[b],
            gsem[b]).wait()

    def start_write(k, b):
        pltpu.async_copy(rows[b], out_hbm.at[pl.ds(base + k * CHUNK, CHUNK)],
                         wsem[b])

    def wait_write(k, b):
        pltpu.make_async_copy(rows[b],
                              out_hbm.at[pl.ds(base + k * CHUNK, CHUNK)],
                              wsem[b]).wait()

    start_gather(0, 0)
    start_gather(1, 1)
    start_gather(2, 2)
    wait_gather(0, 0)
    start_write(0, 0)
    start_gather(3, 3)
    wait_gather(1, 1)
    start_write(1, 1)

    STEADY_END = 2 + ((NCHUNK - 2 - 2) // NBUF) * NBUF

    def body(g, carry):
        k0 = 2 + g * NBUF
        for j in range(NBUF):
            k = k0 + j
            b = (2 + j) % NBUF
            b2 = j % NBUF
            wait_write(k - 2, b2)
            start_gather(k + 2, b2)
            wait_gather(k, b)
            start_write(k, b)
        return carry

    lax.fori_loop(0, (STEADY_END - 2) // NBUF, body, 0)

    for k in range(STEADY_END, NCHUNK):
        b = k % NBUF
        if k + 2 < NCHUNK:
            wait_write(k - 2, (k + 2) % NBUF)
            start_gather(k + 2, (k + 2) % NBUF)
        wait_gather(k, b)
        start_write(k, b)

    for k in range(NCHUNK - NBUF, NCHUNK):
        wait_write(k, k % NBUF)


def kernel(input_ids, table):
    flat = input_ids.reshape(TOT).astype(jnp.int32)
    out = _emb_lookup(flat, table)
    return out.reshape(B, N, DIM)
